# baseline (speedup 1.0000x reference)
import functools

import jax
import jax.numpy as jnp
from jax import lax
from jax.experimental import pallas as pl
from jax.experimental.pallas import tpu as pltpu
from jax.experimental.pallas import tpu_sc as plsc

VOCAB = 8192
B, T = 8, 512
N = B * T
NW = 32
ROWS_PER_W = N // NW
K = 4
NBUF = 2
NCHUNK = ROWS_PER_W // K
NBLK = NCHUNK // NBUF

_mesh = plsc.VectorSubcoreMesh(core_axis_name="c", subcore_axis_name="s")


@functools.partial(
    pl.kernel,
    out_type=jax.ShapeDtypeStruct((N, VOCAB), jnp.float32),
    mesh=_mesh,
    scratch_types=[
        pltpu.VMEM((NCHUNK, K), jnp.int32),
        pltpu.VMEM((NBUF, K, VOCAB), jnp.float32),
        pltpu.SemaphoreType.DMA((NBUF,)),
        pltpu.SemaphoreType.DMA((NBUF,)),
    ],
)
def _gather_sc(idx_hbm, emb_hbm, out_hbm, idx_v, rows_v, gsem, ssem):
    wid = lax.axis_index("s") * 2 + lax.axis_index("c")
    base = wid * ROWS_PER_W
    pltpu.sync_copy(idx_hbm.at[pl.ds(wid * NCHUNK, NCHUNK)], idx_v)

    def gather(c, b):
        pltpu.async_copy(emb_hbm.at[idx_v.at[c]], rows_v.at[b], gsem.at[b])

    def store(c, b):
        pltpu.async_copy(rows_v.at[b], out_hbm.at[pl.ds(base + c * K, K)],
                         ssem.at[b])

    def wait_g(b):
        pltpu.make_async_copy(emb_hbm.at[pl.ds(0, K)], rows_v.at[b],
                              gsem.at[b]).wait()

    def wait_s(b):
        pltpu.make_async_copy(rows_v.at[b], out_hbm.at[pl.ds(base, K)],
                              ssem.at[b]).wait()

    for b in range(NBUF):
        gather(b, b)

    def body(k, carry):
        c0 = k * NBUF
        for b in range(NBUF):
            wait_g(b)
            store(c0 + b, b)
        for b in range(NBUF):
            wait_s(b)
            gather(c0 + NBUF + b, b)
        return carry

    lax.fori_loop(0, NBLK - 1, body, 0)

    c0 = (NBLK - 1) * NBUF
    for b in range(NBUF):
        wait_g(b)
        store(c0 + b, b)
    for b in range(NBUF):
        wait_s(b)


def kernel(x, emb):
    idx2d = x.reshape(NCHUNK * NW, K)
    out = _gather_sc(idx2d, emb)
    return out.reshape(B, T, VOCAB)


# speedup vs baseline: 1.1102x; 1.1102x over previous
import functools

import jax
import jax.numpy as jnp
from jax import lax
from jax.experimental import pallas as pl
from jax.experimental.pallas import tpu as pltpu
from jax.experimental.pallas import tpu_sc as plsc

VOCAB = 8192
N = 4096
NW = 32
NSUB = 16
ROWS_PER_W = N // NW
NBUF = 8
NGRP = ROWS_PER_W // 16

_mesh = plsc.VectorSubcoreMesh(core_axis_name="c", subcore_axis_name="s")


@functools.partial(
    pl.kernel,
    out_type=jax.ShapeDtypeStruct((N, VOCAB), jnp.float32),
    mesh=_mesh,
    scratch_types=[
        pltpu.VMEM((ROWS_PER_W,), jnp.int32),
        pltpu.MemorySpace.VMEM_SHARED((NSUB, NBUF, 1, VOCAB), jnp.float32),
        pltpu.SemaphoreType.DMA((NBUF,)),
        pltpu.SemaphoreType.DMA((NBUF,)),
    ],
)
def _gather_sc(idx_hbm, emb_hbm, out_hbm, idx_v, rows_sh, gsem, ssem):
    sid = lax.axis_index("s")
    wid = sid * 2 + lax.axis_index("c")
    base = wid * ROWS_PER_W
    pltpu.sync_copy(idx_hbm.at[pl.ds(base, ROWS_PER_W)], idx_v)

    def gather(row, b):
        pltpu.async_copy(emb_hbm.at[pl.ds(row, 1)], rows_sh.at[sid, b],
                         gsem.at[b])

    def store(c, b):
        pltpu.async_copy(rows_sh.at[sid, b],
                         out_hbm.at[pl.ds(base + c, 1)], ssem.at[b])

    def wait_g(b):
        pltpu.make_async_copy(emb_hbm.at[pl.ds(0, 1)], rows_sh.at[sid, b],
                              gsem.at[b]).wait()

    def wait_s(b):
        pltpu.make_async_copy(rows_sh.at[sid, b], out_hbm.at[pl.ds(base, 1)],
                              ssem.at[b]).wait()

    def idx_vec(g):
        return idx_v[pl.ds(g * 16, 16)]

    v0 = idx_vec(0)
    for j in range(NBUF):
        gather(v0[j], j)

    def body(g, carry):
        vec = idx_vec(g)
        nxt = idx_vec(g + 1)
        c0 = g * 16
        for j in range(NBUF):
            wait_g(j)
            store(c0 + j, j)
        for j in range(NBUF):
            wait_s(j)
            gather(vec[8 + j], j)
        for j in range(NBUF):
            wait_g(j)
            store(c0 + 8 + j, j)
        for j in range(NBUF):
            wait_s(j)
            gather(nxt[j], j)
        return carry

    lax.fori_loop(0, NGRP - 1, body, 0)

    vec = idx_vec(NGRP - 1)
    c0 = (NGRP - 1) * 16
    for j in range(NBUF):
        wait_g(j)
        store(c0 + j, j)
    for j in range(NBUF):
        wait_s(j)
        gather(vec[8 + j], j)
    for j in range(NBUF):
        wait_g(j)
        store(c0 + 8 + j, j)
    for j in range(NBUF):
        wait_s(j)


def kernel(x, emb):
    out = _gather_sc(x.reshape(N), emb)
    return out.reshape(8, 512, VOCAB)
